# Initial kernel scaffold; baseline (speedup 1.0000x reference)
import jax
import jax.numpy as jnp
from jax.experimental import pallas as pl


def kernel(x, edge_index, params):
    raise NotImplementedError("write your pallas kernel here")


# speedup vs baseline: 3.1943x; 3.1943x over previous
import functools

import jax
import jax.numpy as jnp
from jax import lax
from jax.experimental import pallas as pl
from jax.experimental.pallas import tpu as pltpu
from jax.experimental.pallas import tpu_sc as plsc

N = 10000
E = 320000
D_IN = 128
D_H = 256
D_OUT = 16
NUM_LAYERS = 5

NT = 16
CHUNK = 128
K = 160
EPT = K * CHUNK
EPAD = NT * EPT
NACC = 10240
ROWS_OUT = 632
ROWS_OUT_LAST = N - 15 * ROWS_OUT

DC = 128


@functools.lru_cache(maxsize=None)
def _make_segsum(colsplit):
    KEFF = K if colsplit else K // 2
    KIB = 16
    NSTAGE = KEFF // KIB

    def body(hL, hR, col2, row2, outL, outR, acc, colv, rowv, gbuf, zbuf, sems):
        cid = lax.axis_index("c")
        wid = lax.axis_index("s")

        for r in range(8):
            for jj in range(DC // 16):
                zbuf[r, pl.ds(jj * 16, 16)] = jnp.zeros((16,), jnp.float32)
        zb = wid * (NACC // NT)

        def zero_step(s, _):
            pltpu.sync_copy(zbuf, acc.at[pl.ds(zb + s * 8, 8)])
            return 0

        lax.fori_loop(0, NACC // NT // 8, zero_step, 0)
        plsc.subcore_barrier()

        if colsplit:
            cb = wid * KEFF
        else:
            cb = (cid * NT + wid) * KEFF

        def load_stage(s, sb):
            pltpu.sync_copy(col2.at[pl.ds(cb + s * KIB, KIB)], colv.at[sb])
            pltpu.sync_copy(row2.at[pl.ds(cb + s * KIB, KIB)], rowv.at[sb])

        def run(hsrc):
            load_stage(0, 0)
            pltpu.make_async_copy(hsrc.at[colv.at[0, 0]], gbuf.at[0], sems.at[0]).start()

            def stage(s, _):
                sb = lax.rem(s, 2)

                @pl.when(s + 1 < NSTAGE)
                def _():
                    load_stage(s + 1, 1 - sb)

                def step(j, _):
                    b = lax.rem(j, 2)
                    pltpu.make_async_copy(
                        hsrc.at[colv.at[sb, j]], gbuf.at[b], sems.at[b]
                    ).wait()

                    @pl.when(j + 1 < KIB)
                    def _():
                        pltpu.make_async_copy(
                            hsrc.at[colv.at[sb, j + 1]], gbuf.at[1 - b], sems.at[1 - b]
                        ).start()

                    @pl.when(jnp.logical_and(j + 1 == KIB, s + 1 < NSTAGE))
                    def _():
                        pltpu.make_async_copy(
                            hsrc.at[colv.at[1 - sb, 0]], gbuf.at[1 - b], sems.at[1 - b]
                        ).start()

                    pltpu.sync_copy(gbuf.at[b], acc.at[rowv.at[sb, j]], add=True)
                    return 0

                lax.fori_loop(0, KIB, step, 0)
                return 0

            lax.fori_loop(0, NSTAGE, stage, 0)

        if colsplit:
            @pl.when(cid == 0)
            def _():
                run(hL)

            @pl.when(cid == 1)
            def _():
                run(hR)
        else:
            run(hL)

        plsc.subcore_barrier()
        ob = wid * ROWS_OUT

        def copy_out(dst, nrows):
            pltpu.sync_copy(acc.at[pl.ds(ob, nrows)], dst.at[pl.ds(ob, nrows)])

        @pl.when(jnp.logical_and(cid == 0, wid < 15))
        def _():
            copy_out(outL, ROWS_OUT)

        @pl.when(jnp.logical_and(cid == 0, wid == 15))
        def _():
            copy_out(outL, ROWS_OUT_LAST)

        @pl.when(jnp.logical_and(cid == 1, wid < 15))
        def _():
            copy_out(outR, ROWS_OUT)

        @pl.when(jnp.logical_and(cid == 1, wid == 15))
        def _():
            copy_out(outR, ROWS_OUT_LAST)

    return pl.kernel(
        body,
        out_type=(
            jax.ShapeDtypeStruct((N, DC), jnp.float32),
            jax.ShapeDtypeStruct((N, DC), jnp.float32),
        ),
        mesh=plsc.VectorSubcoreMesh(core_axis_name="c", subcore_axis_name="s"),
        scratch_types=[
            pltpu.VMEM_SHARED((NACC, DC), jnp.float32),
            pltpu.VMEM((2, KIB, CHUNK), jnp.int32),
            pltpu.VMEM((2, KIB, CHUNK), jnp.int32),
            pltpu.VMEM((2, CHUNK, DC), jnp.float32),
            pltpu.VMEM((8, DC), jnp.float32),
            pltpu.SemaphoreType.DMA((2,)),
        ],
    )


RB = 1000
GRID = N // RB


def _t1_body(segL, segR, hL, hR, w1, b1, c, a_out, s1, s2, csx):
    i = pl.program_id(0)
    cc = c[0, 0]
    Dc = hL.shape[1]
    pL = segL[...] + cc * hL[...]
    pR = segR[...] + cc * hR[...]
    a = (
        jnp.dot(pL, w1[:Dc, :], preferred_element_type=jnp.float32)
        + jnp.dot(pR, w1[Dc:, :], preferred_element_type=jnp.float32)
        + b1[...]
    )
    a_out[...] = a
    p1 = jnp.sum(a, axis=0, keepdims=True)
    p2 = jnp.sum(a * a, axis=0, keepdims=True)
    px = jnp.concatenate(
        [jnp.sum(hL[...], axis=0, keepdims=True), jnp.sum(hR[...], axis=0, keepdims=True)],
        axis=1,
    )

    @pl.when(i == 0)
    def _():
        s1[...] = p1
        s2[...] = p2
        csx[...] = px

    @pl.when(i > 0)
    def _():
        s1[...] += p1
        s2[...] += p2
        csx[...] += px


def _t1a_body(seg0, seg1, h, w1, b1, c, a_out, s1, s2, csx):
    i = pl.program_id(0)
    cc = c[0, 0]
    pooled = seg0[...] + seg1[...] + cc * h[...]
    a = jnp.dot(pooled, w1[...], preferred_element_type=jnp.float32) + b1[...]
    a_out[...] = a
    p1 = jnp.sum(a, axis=0, keepdims=True)
    p2 = jnp.sum(a * a, axis=0, keepdims=True)
    px = jnp.sum(h[...], axis=0, keepdims=True)

    @pl.when(i == 0)
    def _():
        s1[...] = p1
        s2[...] = p2
        csx[...] = px

    @pl.when(i > 0)
    def _():
        s1[...] += p1
        s2[...] += p2
        csx[...] += px


def _t2_body(a, s1, s2, g, be, w2, b2, h2_out, u1, u2):
    i = pl.program_id(0)
    mu = s1[...] / N
    var = s2[...] / N - mu * mu
    inv = lax.rsqrt(var + 1e-5) * g[...]
    r = jnp.maximum((a[...] - mu) * inv + be[...], 0.0)
    h2 = jnp.dot(r, w2[...], preferred_element_type=jnp.float32) + b2[...]
    h2_out[...] = h2
    p1 = jnp.sum(h2, axis=0, keepdims=True)
    p2 = jnp.sum(h2 * h2, axis=0, keepdims=True)

    @pl.when(i == 0)
    def _():
        u1[...] = p1
        u2[...] = p2

    @pl.when(i > 0)
    def _():
        u1[...] += p1
        u2[...] += p2


def _t3_body(h2, s1, s2, g, be, hL_out, hR_out, cs):
    i = pl.program_id(0)
    mu = s1[...] / N
    var = s2[...] / N - mu * mu
    inv = lax.rsqrt(var + 1e-5) * g[...]
    h = jnp.maximum((h2[...] - mu) * inv + be[...], 0.0)
    hL_out[...] = h[:, : D_H // 2]
    hR_out[...] = h[:, D_H // 2 :]
    p = jnp.sum(h, axis=0, keepdims=True)

    @pl.when(i == 0)
    def _():
        cs[...] = p

    @pl.when(i > 0)
    def _():
        cs[...] += p


def _row_spec(w):
    return pl.BlockSpec((RB, w), lambda i: (i, 0))


def _full_spec(hw, w):
    return pl.BlockSpec((hw, w), lambda i: (0, 0))


_t1_call = pl.pallas_call(
    _t1_body,
    grid=(GRID,),
    in_specs=[
        _row_spec(128),
        _row_spec(128),
        _row_spec(128),
        _row_spec(128),
        _full_spec(D_H, D_H),
        _full_spec(1, D_H),
        pl.BlockSpec(memory_space=pltpu.SMEM),
    ],
    out_specs=(
        _row_spec(D_H),
        _full_spec(1, D_H),
        _full_spec(1, D_H),
        _full_spec(1, D_H),
    ),
    out_shape=(
        jax.ShapeDtypeStruct((N, D_H), jnp.float32),
        jax.ShapeDtypeStruct((1, D_H), jnp.float32),
        jax.ShapeDtypeStruct((1, D_H), jnp.float32),
        jax.ShapeDtypeStruct((1, D_H), jnp.float32),
    ),
)

_t1a_call = pl.pallas_call(
    _t1a_body,
    grid=(GRID,),
    in_specs=[
        _row_spec(D_IN),
        _row_spec(D_IN),
        _row_spec(D_IN),
        _full_spec(D_IN, D_H),
        _full_spec(1, D_H),
        pl.BlockSpec(memory_space=pltpu.SMEM),
    ],
    out_specs=(
        _row_spec(D_H),
        _full_spec(1, D_H),
        _full_spec(1, D_H),
        _full_spec(1, D_IN),
    ),
    out_shape=(
        jax.ShapeDtypeStruct((N, D_H), jnp.float32),
        jax.ShapeDtypeStruct((1, D_H), jnp.float32),
        jax.ShapeDtypeStruct((1, D_H), jnp.float32),
        jax.ShapeDtypeStruct((1, D_IN), jnp.float32),
    ),
)


_t2_call = pl.pallas_call(
    _t2_body,
    grid=(GRID,),
    in_specs=[
        _row_spec(D_H),
        _full_spec(1, D_H),
        _full_spec(1, D_H),
        _full_spec(1, D_H),
        _full_spec(1, D_H),
        _full_spec(D_H, D_H),
        _full_spec(1, D_H),
    ],
    out_specs=(
        _row_spec(D_H),
        _full_spec(1, D_H),
        _full_spec(1, D_H),
    ),
    out_shape=(
        jax.ShapeDtypeStruct((N, D_H), jnp.float32),
        jax.ShapeDtypeStruct((1, D_H), jnp.float32),
        jax.ShapeDtypeStruct((1, D_H), jnp.float32),
    ),
)

_t3_call = pl.pallas_call(
    _t3_body,
    grid=(GRID,),
    in_specs=[
        _row_spec(D_H),
        _full_spec(1, D_H),
        _full_spec(1, D_H),
        _full_spec(1, D_H),
        _full_spec(1, D_H),
    ],
    out_specs=(
        _row_spec(D_H // 2),
        _row_spec(D_H // 2),
        _full_spec(1, D_H),
    ),
    out_shape=(
        jax.ShapeDtypeStruct((N, D_H // 2), jnp.float32),
        jax.ShapeDtypeStruct((N, D_H // 2), jnp.float32),
        jax.ShapeDtypeStruct((1, D_H), jnp.float32),
    ),
)


def _score_body(*refs):
    out = refs[-1]
    acc = jnp.zeros((1, D_OUT), jnp.float32)
    for l in range(NUM_LAYERS):
        cs = refs[l][...]
        w = refs[NUM_LAYERS + l][...]
        b = refs[2 * NUM_LAYERS + l][...]
        acc = acc + jnp.dot(cs, w, preferred_element_type=jnp.float32) + b
    out[...] = acc


def _score_call(dims):
    in_specs = (
        [_full_spec(1, d) for d in dims]
        + [_full_spec(d, D_OUT) for d in dims]
        + [_full_spec(1, D_OUT) for _ in dims]
    )
    return pl.pallas_call(
        _score_body,
        grid=(1,),
        in_specs=in_specs,
        out_specs=_full_spec(1, D_OUT),
        out_shape=jax.ShapeDtypeStruct((1, D_OUT), jnp.float32),
    )


@jax.jit
def kernel(x, edge_index, params):
    row = edge_index[0]
    col = edge_index[1]
    pad = EPAD - E
    colp = jnp.concatenate([col, jnp.zeros((pad,), jnp.int32)]).reshape(EPAD // CHUNK, CHUNK)
    rowp = jnp.concatenate([row, jnp.full((pad,), N, jnp.int32)]).reshape(EPAD // CHUNK, CHUNK)

    colsums = []
    hL = hR = None
    for l in range(NUM_LAYERS - 1):
        p = params["layers"][l]
        c = (1.0 + params["eps"][l]).reshape(1, 1).astype(jnp.float32)
        if l == 0:
            seg0, seg1 = _make_segsum(False)(x, x, colp, rowp)
            a, s1, s2, csx = _t1a_call(
                seg0, seg1, x, p["W1"], p["b1"].reshape(1, D_H), c
            )
            colsums.append(csx)
        else:
            segL, segR = _make_segsum(True)(hL, hR, colp, rowp)
            a, s1, s2, _ = _t1_call(
                segL, segR, hL, hR, p["W1"], p["b1"].reshape(1, D_H), c
            )
        h2, u1, u2 = _t2_call(
            a, s1, s2,
            p["g1"].reshape(1, D_H), p["be1"].reshape(1, D_H),
            p["W2"], p["b2"].reshape(1, D_H),
        )
        hL, hR, cs = _t3_call(
            h2, u1, u2,
            p["g_out"].reshape(1, D_H), p["be_out"].reshape(1, D_H),
        )
        colsums.append(cs)

    dims = [D_IN] + [D_H] * (NUM_LAYERS - 1)
    ws = [params["preds"][l]["W"] for l in range(NUM_LAYERS)]
    bs = [params["preds"][l]["b"].reshape(1, D_OUT) for l in range(NUM_LAYERS)]
    return _score_call(dims)(*(colsums + ws + bs))


# speedup vs baseline: 3.2257x; 1.0098x over previous
import functools

import jax
import jax.numpy as jnp
from jax import lax
from jax.experimental import pallas as pl
from jax.experimental.pallas import tpu as pltpu
from jax.experimental.pallas import tpu_sc as plsc

N = 10000
E = 320000
D_IN = 128
D_H = 256
D_OUT = 16
NUM_LAYERS = 5

NT = 16
CHUNK = 128
K = 160
EPT = K * CHUNK
EPAD = NT * EPT
NACC = 10240
ROWS_OUT = 632
ROWS_OUT_LAST = N - 15 * ROWS_OUT

DC = 128


@functools.lru_cache(maxsize=None)
def _make_segsum(colsplit):
    KEFF = K if colsplit else K // 2
    KIB = 16
    NSTAGE = KEFF // KIB

    def body(hL, hR, col2, row2, outL, outR, acc, colv, rowv, gbuf, zbuf, sems):
        cid = lax.axis_index("c")
        wid = lax.axis_index("s")

        ZR = 32
        for r in range(ZR):
            for jj in range(DC // 16):
                zbuf[r, pl.ds(jj * 16, 16)] = jnp.zeros((16,), jnp.float32)
        zb = wid * (NACC // NT)
        NZ = NACC // NT // ZR

        def zdesc(s):
            return pltpu.make_async_copy(zbuf, acc.at[pl.ds(zb + s * ZR, ZR)], sems.at[6])

        def zero_fire(s, _):
            zdesc(s).start()
            return 0

        def zero_drain(s, _):
            zdesc(s).wait()
            return 0

        lax.fori_loop(0, NZ, zero_fire, 0)
        lax.fori_loop(0, NZ, zero_drain, 0)
        plsc.subcore_barrier()

        if colsplit:
            cb = wid * KEFF
        else:
            cb = (cid * NT + wid) * KEFF

        def idx_loads(s, sb):
            return (
                pltpu.make_async_copy(
                    col2.at[pl.ds(cb + s * KIB, KIB)], colv.at[sb], sems.at[4]
                ),
                pltpu.make_async_copy(
                    row2.at[pl.ds(cb + s * KIB, KIB)], rowv.at[sb], sems.at[5]
                ),
            )

        def run(hsrc):
            for d in idx_loads(0, 0):
                d.start()
            for d in idx_loads(0, 0):
                d.wait()
            pltpu.make_async_copy(hsrc.at[colv.at[0, 0]], gbuf.at[0], sems.at[0]).start()

            def gdesc(idx_slice, b):
                return pltpu.make_async_copy(hsrc.at[idx_slice], gbuf.at[b], sems.at[b])

            def sdesc(idx_slice, b):
                return pltpu.make_async_copy(gbuf.at[b], acc.at[idx_slice], sems.at[2 + b])

            def stage(s, _):
                sb = lax.rem(s, 2)

                @pl.when(s + 1 < NSTAGE)
                def _():
                    for d in idx_loads(s + 1, 1 - sb):
                        d.start()

                def step(j, _):
                    g = s * KIB + j
                    b = lax.rem(j, 2)
                    gdesc(colv.at[sb, j], b).wait()
                    sdesc(rowv.at[sb, j], b).start(add=True)

                    @pl.when(g >= 1)
                    def _():
                        sdesc(rowv.at[sb, j], 1 - b).wait()

                    @pl.when(j + 1 < KIB)
                    def _():
                        gdesc(colv.at[sb, j + 1], 1 - b).start()

                    @pl.when(jnp.logical_and(j + 1 == KIB, s + 1 < NSTAGE))
                    def _():
                        for d in idx_loads(s + 1, 1 - sb):
                            d.wait()
                        gdesc(colv.at[1 - sb, 0], 1 - b).start()

                    return 0

                lax.fori_loop(0, KIB, step, 0)
                return 0

            lax.fori_loop(0, NSTAGE, stage, 0)
            sdesc(rowv.at[lax.rem(NSTAGE - 1, 2), KIB - 1], (KIB - 1) % 2).wait()

        if colsplit:
            @pl.when(cid == 0)
            def _():
                run(hL)

            @pl.when(cid == 1)
            def _():
                run(hR)
        else:
            run(hL)

        plsc.subcore_barrier()
        ob = wid * ROWS_OUT

        def copy_out(dst, nrows):
            pltpu.sync_copy(acc.at[pl.ds(ob, nrows)], dst.at[pl.ds(ob, nrows)])

        @pl.when(jnp.logical_and(cid == 0, wid < 15))
        def _():
            copy_out(outL, ROWS_OUT)

        @pl.when(jnp.logical_and(cid == 0, wid == 15))
        def _():
            copy_out(outL, ROWS_OUT_LAST)

        @pl.when(jnp.logical_and(cid == 1, wid < 15))
        def _():
            copy_out(outR, ROWS_OUT)

        @pl.when(jnp.logical_and(cid == 1, wid == 15))
        def _():
            copy_out(outR, ROWS_OUT_LAST)

    return pl.kernel(
        body,
        out_type=(
            jax.ShapeDtypeStruct((N, DC), jnp.float32),
            jax.ShapeDtypeStruct((N, DC), jnp.float32),
        ),
        mesh=plsc.VectorSubcoreMesh(core_axis_name="c", subcore_axis_name="s"),
        scratch_types=[
            pltpu.VMEM_SHARED((NACC, DC), jnp.float32),
            pltpu.VMEM((2, KIB, CHUNK), jnp.int32),
            pltpu.VMEM((2, KIB, CHUNK), jnp.int32),
            pltpu.VMEM((2, CHUNK, DC), jnp.float32),
            pltpu.VMEM((32, DC), jnp.float32),
            pltpu.SemaphoreType.DMA((7,)),
        ],
    )


RB = 1000
GRID = N // RB


def _t1_body(segL, segR, hL, hR, w1, b1, c, a_out, s1, s2, csx):
    i = pl.program_id(0)
    cc = c[0, 0]
    Dc = hL.shape[1]
    pL = segL[...] + cc * hL[...]
    pR = segR[...] + cc * hR[...]
    a = (
        jnp.dot(pL, w1[:Dc, :], preferred_element_type=jnp.float32)
        + jnp.dot(pR, w1[Dc:, :], preferred_element_type=jnp.float32)
        + b1[...]
    )
    a_out[...] = a
    p1 = jnp.sum(a, axis=0, keepdims=True)
    p2 = jnp.sum(a * a, axis=0, keepdims=True)
    px = jnp.concatenate(
        [jnp.sum(hL[...], axis=0, keepdims=True), jnp.sum(hR[...], axis=0, keepdims=True)],
        axis=1,
    )

    @pl.when(i == 0)
    def _():
        s1[...] = p1
        s2[...] = p2
        csx[...] = px

    @pl.when(i > 0)
    def _():
        s1[...] += p1
        s2[...] += p2
        csx[...] += px


def _t1a_body(seg0, seg1, h, w1, b1, c, a_out, s1, s2, csx):
    i = pl.program_id(0)
    cc = c[0, 0]
    pooled = seg0[...] + seg1[...] + cc * h[...]
    a = jnp.dot(pooled, w1[...], preferred_element_type=jnp.float32) + b1[...]
    a_out[...] = a
    p1 = jnp.sum(a, axis=0, keepdims=True)
    p2 = jnp.sum(a * a, axis=0, keepdims=True)
    px = jnp.sum(h[...], axis=0, keepdims=True)

    @pl.when(i == 0)
    def _():
        s1[...] = p1
        s2[...] = p2
        csx[...] = px

    @pl.when(i > 0)
    def _():
        s1[...] += p1
        s2[...] += p2
        csx[...] += px


def _t2_body(a, s1, s2, g, be, w2, b2, h2_out, u1, u2):
    i = pl.program_id(0)
    mu = s1[...] / N
    var = s2[...] / N - mu * mu
    inv = lax.rsqrt(var + 1e-5) * g[...]
    r = jnp.maximum((a[...] - mu) * inv + be[...], 0.0)
    h2 = jnp.dot(r, w2[...], preferred_element_type=jnp.float32) + b2[...]
    h2_out[...] = h2
    p1 = jnp.sum(h2, axis=0, keepdims=True)
    p2 = jnp.sum(h2 * h2, axis=0, keepdims=True)

    @pl.when(i == 0)
    def _():
        u1[...] = p1
        u2[...] = p2

    @pl.when(i > 0)
    def _():
        u1[...] += p1
        u2[...] += p2


def _t3_body(h2, s1, s2, g, be, hL_out, hR_out, cs):
    i = pl.program_id(0)
    mu = s1[...] / N
    var = s2[...] / N - mu * mu
    inv = lax.rsqrt(var + 1e-5) * g[...]
    h = jnp.maximum((h2[...] - mu) * inv + be[...], 0.0)
    hL_out[...] = h[:, : D_H // 2]
    hR_out[...] = h[:, D_H // 2 :]
    p = jnp.sum(h, axis=0, keepdims=True)

    @pl.when(i == 0)
    def _():
        cs[...] = p

    @pl.when(i > 0)
    def _():
        cs[...] += p


def _row_spec(w):
    return pl.BlockSpec((RB, w), lambda i: (i, 0))


def _full_spec(hw, w):
    return pl.BlockSpec((hw, w), lambda i: (0, 0))


_t1_call = pl.pallas_call(
    _t1_body,
    grid=(GRID,),
    in_specs=[
        _row_spec(128),
        _row_spec(128),
        _row_spec(128),
        _row_spec(128),
        _full_spec(D_H, D_H),
        _full_spec(1, D_H),
        pl.BlockSpec(memory_space=pltpu.SMEM),
    ],
    out_specs=(
        _row_spec(D_H),
        _full_spec(1, D_H),
        _full_spec(1, D_H),
        _full_spec(1, D_H),
    ),
    out_shape=(
        jax.ShapeDtypeStruct((N, D_H), jnp.float32),
        jax.ShapeDtypeStruct((1, D_H), jnp.float32),
        jax.ShapeDtypeStruct((1, D_H), jnp.float32),
        jax.ShapeDtypeStruct((1, D_H), jnp.float32),
    ),
)

_t1a_call = pl.pallas_call(
    _t1a_body,
    grid=(GRID,),
    in_specs=[
        _row_spec(D_IN),
        _row_spec(D_IN),
        _row_spec(D_IN),
        _full_spec(D_IN, D_H),
        _full_spec(1, D_H),
        pl.BlockSpec(memory_space=pltpu.SMEM),
    ],
    out_specs=(
        _row_spec(D_H),
        _full_spec(1, D_H),
        _full_spec(1, D_H),
        _full_spec(1, D_IN),
    ),
    out_shape=(
        jax.ShapeDtypeStruct((N, D_H), jnp.float32),
        jax.ShapeDtypeStruct((1, D_H), jnp.float32),
        jax.ShapeDtypeStruct((1, D_H), jnp.float32),
        jax.ShapeDtypeStruct((1, D_IN), jnp.float32),
    ),
)


_t2_call = pl.pallas_call(
    _t2_body,
    grid=(GRID,),
    in_specs=[
        _row_spec(D_H),
        _full_spec(1, D_H),
        _full_spec(1, D_H),
        _full_spec(1, D_H),
        _full_spec(1, D_H),
        _full_spec(D_H, D_H),
        _full_spec(1, D_H),
    ],
    out_specs=(
        _row_spec(D_H),
        _full_spec(1, D_H),
        _full_spec(1, D_H),
    ),
    out_shape=(
        jax.ShapeDtypeStruct((N, D_H), jnp.float32),
        jax.ShapeDtypeStruct((1, D_H), jnp.float32),
        jax.ShapeDtypeStruct((1, D_H), jnp.float32),
    ),
)

_t3_call = pl.pallas_call(
    _t3_body,
    grid=(GRID,),
    in_specs=[
        _row_spec(D_H),
        _full_spec(1, D_H),
        _full_spec(1, D_H),
        _full_spec(1, D_H),
        _full_spec(1, D_H),
    ],
    out_specs=(
        _row_spec(D_H // 2),
        _row_spec(D_H // 2),
        _full_spec(1, D_H),
    ),
    out_shape=(
        jax.ShapeDtypeStruct((N, D_H // 2), jnp.float32),
        jax.ShapeDtypeStruct((N, D_H // 2), jnp.float32),
        jax.ShapeDtypeStruct((1, D_H), jnp.float32),
    ),
)


def _score_body(*refs):
    out = refs[-1]
    acc = jnp.zeros((1, D_OUT), jnp.float32)
    for l in range(NUM_LAYERS):
        cs = refs[l][...]
        w = refs[NUM_LAYERS + l][...]
        b = refs[2 * NUM_LAYERS + l][...]
        acc = acc + jnp.dot(cs, w, preferred_element_type=jnp.float32) + b
    out[...] = acc


def _score_call(dims):
    in_specs = (
        [_full_spec(1, d) for d in dims]
        + [_full_spec(d, D_OUT) for d in dims]
        + [_full_spec(1, D_OUT) for _ in dims]
    )
    return pl.pallas_call(
        _score_body,
        grid=(1,),
        in_specs=in_specs,
        out_specs=_full_spec(1, D_OUT),
        out_shape=jax.ShapeDtypeStruct((1, D_OUT), jnp.float32),
    )


@jax.jit
def kernel(x, edge_index, params):
    row = edge_index[0]
    col = edge_index[1]
    pad = EPAD - E
    colp = jnp.concatenate([col, jnp.zeros((pad,), jnp.int32)]).reshape(EPAD // CHUNK, CHUNK)
    rowp = jnp.concatenate([row, jnp.full((pad,), N, jnp.int32)]).reshape(EPAD // CHUNK, CHUNK)

    colsums = []
    hL = hR = None
    for l in range(NUM_LAYERS - 1):
        p = params["layers"][l]
        c = (1.0 + params["eps"][l]).reshape(1, 1).astype(jnp.float32)
        if l == 0:
            seg0, seg1 = _make_segsum(False)(x, x, colp, rowp)
            a, s1, s2, csx = _t1a_call(
                seg0, seg1, x, p["W1"], p["b1"].reshape(1, D_H), c
            )
            colsums.append(csx)
        else:
            segL, segR = _make_segsum(True)(hL, hR, colp, rowp)
            a, s1, s2, _ = _t1_call(
                segL, segR, hL, hR, p["W1"], p["b1"].reshape(1, D_H), c
            )
        h2, u1, u2 = _t2_call(
            a, s1, s2,
            p["g1"].reshape(1, D_H), p["be1"].reshape(1, D_H),
            p["W2"], p["b2"].reshape(1, D_H),
        )
        hL, hR, cs = _t3_call(
            h2, u1, u2,
            p["g_out"].reshape(1, D_H), p["be_out"].reshape(1, D_H),
        )
        colsums.append(cs)

    dims = [D_IN] + [D_H] * (NUM_LAYERS - 1)
    ws = [params["preds"][l]["W"] for l in range(NUM_LAYERS)]
    bs = [params["preds"][l]["b"].reshape(1, D_OUT) for l in range(NUM_LAYERS)]
    return _score_call(dims)(*(colsums + ws + bs))


# speedup vs baseline: 3.2329x; 1.0022x over previous
import functools

import jax
import jax.numpy as jnp
from jax import lax
from jax.experimental import pallas as pl
from jax.experimental.pallas import tpu as pltpu
from jax.experimental.pallas import tpu_sc as plsc

N = 10000
E = 320000
D_IN = 128
D_H = 256
D_OUT = 16
NUM_LAYERS = 5

NT = 16
CHUNK = 64
K = 320
NB = 4
EPT = K * CHUNK
EPAD = NT * EPT
NACC = 10240
ROWS_OUT = 632
ROWS_OUT_LAST = N - 15 * ROWS_OUT

DC = 128


@functools.lru_cache(maxsize=None)
def _make_segsum(colsplit):
    KEFF = K if colsplit else K // 2
    KIB = 16
    NSTAGE = KEFF // KIB

    def body(hL, hR, col2, row2, outL, outR, acc, colv, rowv, gbuf, zbuf, sems):
        cid = lax.axis_index("c")
        wid = lax.axis_index("s")

        ZR = 32
        for r in range(ZR):
            for jj in range(DC // 16):
                zbuf[r, pl.ds(jj * 16, 16)] = jnp.zeros((16,), jnp.float32)
        zb = wid * (NACC // NT)
        NZ = NACC // NT // ZR

        def zdesc(s):
            return pltpu.make_async_copy(zbuf, acc.at[pl.ds(zb + s * ZR, ZR)], sems.at[2 * NB + 2])

        def zero_fire(s, _):
            zdesc(s).start()
            return 0

        def zero_drain(s, _):
            zdesc(s).wait()
            return 0

        lax.fori_loop(0, NZ, zero_fire, 0)
        lax.fori_loop(0, NZ, zero_drain, 0)
        plsc.subcore_barrier()

        if colsplit:
            cb = wid * KEFF
        else:
            cb = (cid * NT + wid) * KEFF

        def idx_loads(s, sb):
            return (
                pltpu.make_async_copy(
                    col2.at[pl.ds(cb + s * KIB, KIB)], colv.at[sb], sems.at[2 * NB]
                ),
                pltpu.make_async_copy(
                    row2.at[pl.ds(cb + s * KIB, KIB)], rowv.at[sb], sems.at[2 * NB + 1]
                ),
            )

        def run(hsrc):
            def gdesc(idx_slice, b):
                return pltpu.make_async_copy(hsrc.at[idx_slice], gbuf.at[b], sems.at[b])

            def sdesc(idx_slice, b):
                return pltpu.make_async_copy(gbuf.at[b], acc.at[idx_slice], sems.at[NB + b])

            for d in idx_loads(0, 0):
                d.start()
            for d in idx_loads(0, 0):
                d.wait()
            for g0 in range(NB - 1):
                gdesc(colv.at[0, g0], g0).start()

            def step(g, _):
                s = lax.div(g, KIB)
                j = lax.rem(g, KIB)
                b = lax.rem(g, NB)

                @pl.when(jnp.logical_and(j == 0, g >= 1))
                def _():
                    sdesc(rowv.at[lax.rem(s, 2), j], lax.rem(g - 1, NB)).wait()

                @pl.when(jnp.logical_and(j == 0, s + 1 < NSTAGE))
                def _():
                    for d in idx_loads(s + 1, lax.rem(s + 1, 2)):
                        d.start()

                gdesc(colv.at[lax.rem(s, 2), j], b).wait()
                sdesc(rowv.at[lax.rem(s, 2), j], b).start(add=True)

                @pl.when(jnp.logical_and(j > 0, g >= 1))
                def _():
                    sdesc(rowv.at[lax.rem(s, 2), j], lax.rem(g - 1, NB)).wait()

                gn = g + NB - 1
                sn = lax.div(gn, KIB)
                jn = lax.rem(gn, KIB)

                @pl.when(jnp.logical_and(jn == 0, gn < KEFF))
                def _():
                    for d in idx_loads(sn, lax.rem(sn, 2)):
                        d.wait()

                @pl.when(gn < KEFF)
                def _():
                    gdesc(colv.at[lax.rem(sn, 2), jn], lax.rem(gn, NB)).start()

                return 0

            lax.fori_loop(0, KEFF, step, 0)
            sdesc(rowv.at[lax.rem(NSTAGE - 1, 2), KIB - 1], lax.rem(KEFF - 1, NB)).wait()

        if colsplit:
            @pl.when(cid == 0)
            def _():
                run(hL)

            @pl.when(cid == 1)
            def _():
                run(hR)
        else:
            run(hL)

        plsc.subcore_barrier()
        ob = wid * ROWS_OUT

        def copy_out(dst, nrows):
            pltpu.sync_copy(acc.at[pl.ds(ob, nrows)], dst.at[pl.ds(ob, nrows)])

        @pl.when(jnp.logical_and(cid == 0, wid < 15))
        def _():
            copy_out(outL, ROWS_OUT)

        @pl.when(jnp.logical_and(cid == 0, wid == 15))
        def _():
            copy_out(outL, ROWS_OUT_LAST)

        @pl.when(jnp.logical_and(cid == 1, wid < 15))
        def _():
            copy_out(outR, ROWS_OUT)

        @pl.when(jnp.logical_and(cid == 1, wid == 15))
        def _():
            copy_out(outR, ROWS_OUT_LAST)

    return pl.kernel(
        body,
        out_type=(
            jax.ShapeDtypeStruct((N, DC), jnp.float32),
            jax.ShapeDtypeStruct((N, DC), jnp.float32),
        ),
        mesh=plsc.VectorSubcoreMesh(core_axis_name="c", subcore_axis_name="s"),
        scratch_types=[
            pltpu.VMEM_SHARED((NACC, DC), jnp.float32),
            pltpu.VMEM((2, KIB, CHUNK), jnp.int32),
            pltpu.VMEM((2, KIB, CHUNK), jnp.int32),
            pltpu.VMEM((NB, CHUNK, DC), jnp.float32),
            pltpu.VMEM((32, DC), jnp.float32),
            pltpu.SemaphoreType.DMA((2 * NB + 3,)),
        ],
    )


RB = 1000
GRID = N // RB


def _t1_body(segL, segR, hL, hR, w1, b1, c, a_out, s1, s2, csx):
    i = pl.program_id(0)
    cc = c[0, 0]
    Dc = hL.shape[1]
    pL = segL[...] + cc * hL[...]
    pR = segR[...] + cc * hR[...]
    a = (
        jnp.dot(pL, w1[:Dc, :], preferred_element_type=jnp.float32)
        + jnp.dot(pR, w1[Dc:, :], preferred_element_type=jnp.float32)
        + b1[...]
    )
    a_out[...] = a
    p1 = jnp.sum(a, axis=0, keepdims=True)
    p2 = jnp.sum(a * a, axis=0, keepdims=True)
    px = jnp.concatenate(
        [jnp.sum(hL[...], axis=0, keepdims=True), jnp.sum(hR[...], axis=0, keepdims=True)],
        axis=1,
    )

    @pl.when(i == 0)
    def _():
        s1[...] = p1
        s2[...] = p2
        csx[...] = px

    @pl.when(i > 0)
    def _():
        s1[...] += p1
        s2[...] += p2
        csx[...] += px


def _t1a_body(seg0, seg1, h, w1, b1, c, a_out, s1, s2, csx):
    i = pl.program_id(0)
    cc = c[0, 0]
    pooled = seg0[...] + seg1[...] + cc * h[...]
    a = jnp.dot(pooled, w1[...], preferred_element_type=jnp.float32) + b1[...]
    a_out[...] = a
    p1 = jnp.sum(a, axis=0, keepdims=True)
    p2 = jnp.sum(a * a, axis=0, keepdims=True)
    px = jnp.sum(h[...], axis=0, keepdims=True)

    @pl.when(i == 0)
    def _():
        s1[...] = p1
        s2[...] = p2
        csx[...] = px

    @pl.when(i > 0)
    def _():
        s1[...] += p1
        s2[...] += p2
        csx[...] += px


def _t2_body(a, s1, s2, g, be, w2, b2, h2_out, u1, u2):
    i = pl.program_id(0)
    mu = s1[...] / N
    var = s2[...] / N - mu * mu
    inv = lax.rsqrt(var + 1e-5) * g[...]
    r = jnp.maximum((a[...] - mu) * inv + be[...], 0.0)
    h2 = jnp.dot(r, w2[...], preferred_element_type=jnp.float32) + b2[...]
    h2_out[...] = h2
    p1 = jnp.sum(h2, axis=0, keepdims=True)
    p2 = jnp.sum(h2 * h2, axis=0, keepdims=True)

    @pl.when(i == 0)
    def _():
        u1[...] = p1
        u2[...] = p2

    @pl.when(i > 0)
    def _():
        u1[...] += p1
        u2[...] += p2


def _t3_body(h2, s1, s2, g, be, hL_out, hR_out, cs):
    i = pl.program_id(0)
    mu = s1[...] / N
    var = s2[...] / N - mu * mu
    inv = lax.rsqrt(var + 1e-5) * g[...]
    h = jnp.maximum((h2[...] - mu) * inv + be[...], 0.0)
    hL_out[...] = h[:, : D_H // 2]
    hR_out[...] = h[:, D_H // 2 :]
    p = jnp.sum(h, axis=0, keepdims=True)

    @pl.when(i == 0)
    def _():
        cs[...] = p

    @pl.when(i > 0)
    def _():
        cs[...] += p


def _row_spec(w):
    return pl.BlockSpec((RB, w), lambda i: (i, 0))


def _full_spec(hw, w):
    return pl.BlockSpec((hw, w), lambda i: (0, 0))


_t1_call = pl.pallas_call(
    _t1_body,
    grid=(GRID,),
    in_specs=[
        _row_spec(128),
        _row_spec(128),
        _row_spec(128),
        _row_spec(128),
        _full_spec(D_H, D_H),
        _full_spec(1, D_H),
        pl.BlockSpec(memory_space=pltpu.SMEM),
    ],
    out_specs=(
        _row_spec(D_H),
        _full_spec(1, D_H),
        _full_spec(1, D_H),
        _full_spec(1, D_H),
    ),
    out_shape=(
        jax.ShapeDtypeStruct((N, D_H), jnp.float32),
        jax.ShapeDtypeStruct((1, D_H), jnp.float32),
        jax.ShapeDtypeStruct((1, D_H), jnp.float32),
        jax.ShapeDtypeStruct((1, D_H), jnp.float32),
    ),
)

_t1a_call = pl.pallas_call(
    _t1a_body,
    grid=(GRID,),
    in_specs=[
        _row_spec(D_IN),
        _row_spec(D_IN),
        _row_spec(D_IN),
        _full_spec(D_IN, D_H),
        _full_spec(1, D_H),
        pl.BlockSpec(memory_space=pltpu.SMEM),
    ],
    out_specs=(
        _row_spec(D_H),
        _full_spec(1, D_H),
        _full_spec(1, D_H),
        _full_spec(1, D_IN),
    ),
    out_shape=(
        jax.ShapeDtypeStruct((N, D_H), jnp.float32),
        jax.ShapeDtypeStruct((1, D_H), jnp.float32),
        jax.ShapeDtypeStruct((1, D_H), jnp.float32),
        jax.ShapeDtypeStruct((1, D_IN), jnp.float32),
    ),
)


_t2_call = pl.pallas_call(
    _t2_body,
    grid=(GRID,),
    in_specs=[
        _row_spec(D_H),
        _full_spec(1, D_H),
        _full_spec(1, D_H),
        _full_spec(1, D_H),
        _full_spec(1, D_H),
        _full_spec(D_H, D_H),
        _full_spec(1, D_H),
    ],
    out_specs=(
        _row_spec(D_H),
        _full_spec(1, D_H),
        _full_spec(1, D_H),
    ),
    out_shape=(
        jax.ShapeDtypeStruct((N, D_H), jnp.float32),
        jax.ShapeDtypeStruct((1, D_H), jnp.float32),
        jax.ShapeDtypeStruct((1, D_H), jnp.float32),
    ),
)

_t3_call = pl.pallas_call(
    _t3_body,
    grid=(GRID,),
    in_specs=[
        _row_spec(D_H),
        _full_spec(1, D_H),
        _full_spec(1, D_H),
        _full_spec(1, D_H),
        _full_spec(1, D_H),
    ],
    out_specs=(
        _row_spec(D_H // 2),
        _row_spec(D_H // 2),
        _full_spec(1, D_H),
    ),
    out_shape=(
        jax.ShapeDtypeStruct((N, D_H // 2), jnp.float32),
        jax.ShapeDtypeStruct((N, D_H // 2), jnp.float32),
        jax.ShapeDtypeStruct((1, D_H), jnp.float32),
    ),
)


def _score_body(*refs):
    out = refs[-1]
    acc = jnp.zeros((1, D_OUT), jnp.float32)
    for l in range(NUM_LAYERS):
        cs = refs[l][...]
        w = refs[NUM_LAYERS + l][...]
        b = refs[2 * NUM_LAYERS + l][...]
        acc = acc + jnp.dot(cs, w, preferred_element_type=jnp.float32) + b
    out[...] = acc


def _score_call(dims):
    in_specs = (
        [_full_spec(1, d) for d in dims]
        + [_full_spec(d, D_OUT) for d in dims]
        + [_full_spec(1, D_OUT) for _ in dims]
    )
    return pl.pallas_call(
        _score_body,
        grid=(1,),
        in_specs=in_specs,
        out_specs=_full_spec(1, D_OUT),
        out_shape=jax.ShapeDtypeStruct((1, D_OUT), jnp.float32),
    )


@jax.jit
def kernel(x, edge_index, params):
    row = edge_index[0]
    col = edge_index[1]
    pad = EPAD - E
    colp = jnp.concatenate([col, jnp.zeros((pad,), jnp.int32)]).reshape(EPAD // CHUNK, CHUNK)
    rowp = jnp.concatenate([row, jnp.full((pad,), N, jnp.int32)]).reshape(EPAD // CHUNK, CHUNK)

    colsums = []
    hL = hR = None
    for l in range(NUM_LAYERS - 1):
        p = params["layers"][l]
        c = (1.0 + params["eps"][l]).reshape(1, 1).astype(jnp.float32)
        if l == 0:
            seg0, seg1 = _make_segsum(False)(x, x, colp, rowp)
            a, s1, s2, csx = _t1a_call(
                seg0, seg1, x, p["W1"], p["b1"].reshape(1, D_H), c
            )
            colsums.append(csx)
        else:
            segL, segR = _make_segsum(True)(hL, hR, colp, rowp)
            a, s1, s2, _ = _t1_call(
                segL, segR, hL, hR, p["W1"], p["b1"].reshape(1, D_H), c
            )
        h2, u1, u2 = _t2_call(
            a, s1, s2,
            p["g1"].reshape(1, D_H), p["be1"].reshape(1, D_H),
            p["W2"], p["b2"].reshape(1, D_H),
        )
        hL, hR, cs = _t3_call(
            h2, u1, u2,
            p["g_out"].reshape(1, D_H), p["be_out"].reshape(1, D_H),
        )
        colsums.append(cs)

    dims = [D_IN] + [D_H] * (NUM_LAYERS - 1)
    ws = [params["preds"][l]["W"] for l in range(NUM_LAYERS)]
    bs = [params["preds"][l]["b"].reshape(1, D_OUT) for l in range(NUM_LAYERS)]
    return _score_call(dims)(*(colsums + ws + bs))


# speedup vs baseline: 3.2622x; 1.0091x over previous
import functools

import jax
import jax.numpy as jnp
from jax import lax
from jax.experimental import pallas as pl
from jax.experimental.pallas import tpu as pltpu
from jax.experimental.pallas import tpu_sc as plsc

N = 10000
E = 320000
D_IN = 128
D_H = 256
D_OUT = 16
NUM_LAYERS = 5

NT = 16
CHUNK = 64
K = 320
NB = 4
EPT = K * CHUNK
EPAD = NT * EPT
NACC = 10240
ROWS_OUT = 632
ROWS_OUT_LAST = N - 15 * ROWS_OUT

DC = 128


@functools.lru_cache(maxsize=None)
def _make_segsum(colsplit):
    KEFF = K if colsplit else K // 2
    KIB = 16
    NSTAGE = KEFF // KIB

    def body(hL, hR, col2, row2, outL, outR, acc, colv, rowv, gbuf, zbuf, sems):
        cid = lax.axis_index("c")
        wid = lax.axis_index("s")

        ZR = 32
        for r in range(ZR):
            for jj in range(DC // 16):
                zbuf[r, pl.ds(jj * 16, 16)] = jnp.zeros((16,), jnp.float32)
        zb = wid * (NACC // NT)
        NZ = NACC // NT // ZR

        def zdesc(s):
            return pltpu.make_async_copy(zbuf, acc.at[pl.ds(zb + s * ZR, ZR)], sems.at[2 * NB + 2])

        def zero_fire(s, _):
            zdesc(s).start()
            return 0

        def zero_drain(s, _):
            zdesc(s).wait()
            return 0

        lax.fori_loop(0, NZ, zero_fire, 0)
        lax.fori_loop(0, NZ, zero_drain, 0)
        plsc.subcore_barrier()

        if colsplit:
            cb = wid * KEFF
        else:
            cb = (cid * NT + wid) * KEFF

        def idx_loads(s, sb):
            return (
                pltpu.make_async_copy(
                    col2.at[pl.ds(cb + s * KIB, KIB)], colv.at[sb], sems.at[2 * NB]
                ),
                pltpu.make_async_copy(
                    row2.at[pl.ds(cb + s * KIB, KIB)], rowv.at[sb], sems.at[2 * NB + 1]
                ),
            )

        def run(hsrc):
            def gdesc(idx_slice, b):
                return pltpu.make_async_copy(hsrc.at[idx_slice], gbuf.at[b], sems.at[b])

            def sdesc(idx_slice, b):
                return pltpu.make_async_copy(gbuf.at[b], acc.at[idx_slice], sems.at[NB + b])

            for d in idx_loads(0, 0):
                d.start()
            for d in idx_loads(0, 0):
                d.wait()
            for g0 in range(NB - 1):
                gdesc(colv.at[0, g0], g0).start()

            def step(g, _):
                s = lax.div(g, KIB)
                j = lax.rem(g, KIB)
                b = lax.rem(g, NB)

                @pl.when(jnp.logical_and(j == 0, g >= 1))
                def _():
                    sdesc(rowv.at[lax.rem(s, 2), j], lax.rem(g - 1, NB)).wait()

                @pl.when(jnp.logical_and(j == 0, s + 1 < NSTAGE))
                def _():
                    for d in idx_loads(s + 1, lax.rem(s + 1, 2)):
                        d.start()

                gdesc(colv.at[lax.rem(s, 2), j], b).wait()
                sdesc(rowv.at[lax.rem(s, 2), j], b).start(add=True)

                @pl.when(jnp.logical_and(j > 0, g >= 1))
                def _():
                    sdesc(rowv.at[lax.rem(s, 2), j], lax.rem(g - 1, NB)).wait()

                gn = g + NB - 1
                sn = lax.div(gn, KIB)
                jn = lax.rem(gn, KIB)

                @pl.when(jnp.logical_and(jn == 0, gn < KEFF))
                def _():
                    for d in idx_loads(sn, lax.rem(sn, 2)):
                        d.wait()

                @pl.when(gn < KEFF)
                def _():
                    gdesc(colv.at[lax.rem(sn, 2), jn], lax.rem(gn, NB)).start()

                return 0

            lax.fori_loop(0, KEFF, step, 0)
            sdesc(rowv.at[lax.rem(NSTAGE - 1, 2), KIB - 1], lax.rem(KEFF - 1, NB)).wait()

        if colsplit:
            @pl.when(cid == 0)
            def _():
                run(hL)

            @pl.when(cid == 1)
            def _():
                run(hR)
        else:
            run(hL)

        plsc.subcore_barrier()
        ob = wid * ROWS_OUT

        def copy_out(dst, nrows):
            pltpu.sync_copy(acc.at[pl.ds(ob, nrows)], dst.at[pl.ds(ob, nrows)])

        @pl.when(jnp.logical_and(cid == 0, wid < 15))
        def _():
            copy_out(outL, ROWS_OUT)

        @pl.when(jnp.logical_and(cid == 0, wid == 15))
        def _():
            copy_out(outL, ROWS_OUT_LAST)

        @pl.when(jnp.logical_and(cid == 1, wid < 15))
        def _():
            copy_out(outR, ROWS_OUT)

        @pl.when(jnp.logical_and(cid == 1, wid == 15))
        def _():
            copy_out(outR, ROWS_OUT_LAST)

    return pl.kernel(
        body,
        out_type=(
            jax.ShapeDtypeStruct((N, DC), jnp.float32),
            jax.ShapeDtypeStruct((N, DC), jnp.float32),
        ),
        mesh=plsc.VectorSubcoreMesh(core_axis_name="c", subcore_axis_name="s"),
        scratch_types=[
            pltpu.VMEM_SHARED((NACC, DC), jnp.float32),
            pltpu.VMEM((2, KIB, CHUNK), jnp.int32),
            pltpu.VMEM((2, KIB, CHUNK), jnp.int32),
            pltpu.VMEM((NB, CHUNK, DC), jnp.float32),
            pltpu.VMEM((32, DC), jnp.float32),
            pltpu.SemaphoreType.DMA((2 * NB + 3,)),
        ],
    )


RB = 2000
GRID = N // RB


def _t1_body(segL, segR, hL, hR, w1, b1, c, a_out, s1, s2, csx):
    i = pl.program_id(0)
    cc = c[0, 0]
    Dc = hL.shape[1]
    pL = segL[...] + cc * hL[...]
    pR = segR[...] + cc * hR[...]
    a = (
        jnp.dot(pL, w1[:Dc, :], preferred_element_type=jnp.float32)
        + jnp.dot(pR, w1[Dc:, :], preferred_element_type=jnp.float32)
        + b1[...]
    )
    a_out[...] = a
    p1 = jnp.sum(a, axis=0, keepdims=True)
    p2 = jnp.sum(a * a, axis=0, keepdims=True)
    px = jnp.concatenate(
        [jnp.sum(hL[...], axis=0, keepdims=True), jnp.sum(hR[...], axis=0, keepdims=True)],
        axis=1,
    )

    @pl.when(i == 0)
    def _():
        s1[...] = p1
        s2[...] = p2
        csx[...] = px

    @pl.when(i > 0)
    def _():
        s1[...] += p1
        s2[...] += p2
        csx[...] += px


def _t1a_body(seg0, seg1, h, w1, b1, c, a_out, s1, s2, csx):
    i = pl.program_id(0)
    cc = c[0, 0]
    pooled = seg0[...] + seg1[...] + cc * h[...]
    a = jnp.dot(pooled, w1[...], preferred_element_type=jnp.float32) + b1[...]
    a_out[...] = a
    p1 = jnp.sum(a, axis=0, keepdims=True)
    p2 = jnp.sum(a * a, axis=0, keepdims=True)
    px = jnp.sum(h[...], axis=0, keepdims=True)

    @pl.when(i == 0)
    def _():
        s1[...] = p1
        s2[...] = p2
        csx[...] = px

    @pl.when(i > 0)
    def _():
        s1[...] += p1
        s2[...] += p2
        csx[...] += px


def _t2_body(a, s1, s2, g, be, w2, b2, h2_out, u1, u2):
    i = pl.program_id(0)
    mu = s1[...] / N
    var = s2[...] / N - mu * mu
    inv = lax.rsqrt(var + 1e-5) * g[...]
    r = jnp.maximum((a[...] - mu) * inv + be[...], 0.0)
    h2 = jnp.dot(r, w2[...], preferred_element_type=jnp.float32) + b2[...]
    h2_out[...] = h2
    p1 = jnp.sum(h2, axis=0, keepdims=True)
    p2 = jnp.sum(h2 * h2, axis=0, keepdims=True)

    @pl.when(i == 0)
    def _():
        u1[...] = p1
        u2[...] = p2

    @pl.when(i > 0)
    def _():
        u1[...] += p1
        u2[...] += p2


def _t3_body(h2, s1, s2, g, be, hL_out, hR_out, cs):
    i = pl.program_id(0)
    mu = s1[...] / N
    var = s2[...] / N - mu * mu
    inv = lax.rsqrt(var + 1e-5) * g[...]
    h = jnp.maximum((h2[...] - mu) * inv + be[...], 0.0)
    hL_out[...] = h[:, : D_H // 2]
    hR_out[...] = h[:, D_H // 2 :]
    p = jnp.sum(h, axis=0, keepdims=True)

    @pl.when(i == 0)
    def _():
        cs[...] = p

    @pl.when(i > 0)
    def _():
        cs[...] += p


def _row_spec(w):
    return pl.BlockSpec((RB, w), lambda i: (i, 0))


def _full_spec(hw, w):
    return pl.BlockSpec((hw, w), lambda i: (0, 0))


_t1_call = pl.pallas_call(
    _t1_body,
    grid=(GRID,),
    in_specs=[
        _row_spec(128),
        _row_spec(128),
        _row_spec(128),
        _row_spec(128),
        _full_spec(D_H, D_H),
        _full_spec(1, D_H),
        pl.BlockSpec(memory_space=pltpu.SMEM),
    ],
    out_specs=(
        _row_spec(D_H),
        _full_spec(1, D_H),
        _full_spec(1, D_H),
        _full_spec(1, D_H),
    ),
    out_shape=(
        jax.ShapeDtypeStruct((N, D_H), jnp.float32),
        jax.ShapeDtypeStruct((1, D_H), jnp.float32),
        jax.ShapeDtypeStruct((1, D_H), jnp.float32),
        jax.ShapeDtypeStruct((1, D_H), jnp.float32),
    ),
)

_t1a_call = pl.pallas_call(
    _t1a_body,
    grid=(GRID,),
    in_specs=[
        _row_spec(D_IN),
        _row_spec(D_IN),
        _row_spec(D_IN),
        _full_spec(D_IN, D_H),
        _full_spec(1, D_H),
        pl.BlockSpec(memory_space=pltpu.SMEM),
    ],
    out_specs=(
        _row_spec(D_H),
        _full_spec(1, D_H),
        _full_spec(1, D_H),
        _full_spec(1, D_IN),
    ),
    out_shape=(
        jax.ShapeDtypeStruct((N, D_H), jnp.float32),
        jax.ShapeDtypeStruct((1, D_H), jnp.float32),
        jax.ShapeDtypeStruct((1, D_H), jnp.float32),
        jax.ShapeDtypeStruct((1, D_IN), jnp.float32),
    ),
)


_t2_call = pl.pallas_call(
    _t2_body,
    grid=(GRID,),
    in_specs=[
        _row_spec(D_H),
        _full_spec(1, D_H),
        _full_spec(1, D_H),
        _full_spec(1, D_H),
        _full_spec(1, D_H),
        _full_spec(D_H, D_H),
        _full_spec(1, D_H),
    ],
    out_specs=(
        _row_spec(D_H),
        _full_spec(1, D_H),
        _full_spec(1, D_H),
    ),
    out_shape=(
        jax.ShapeDtypeStruct((N, D_H), jnp.float32),
        jax.ShapeDtypeStruct((1, D_H), jnp.float32),
        jax.ShapeDtypeStruct((1, D_H), jnp.float32),
    ),
)

_t3_call = pl.pallas_call(
    _t3_body,
    grid=(GRID,),
    in_specs=[
        _row_spec(D_H),
        _full_spec(1, D_H),
        _full_spec(1, D_H),
        _full_spec(1, D_H),
        _full_spec(1, D_H),
    ],
    out_specs=(
        _row_spec(D_H // 2),
        _row_spec(D_H // 2),
        _full_spec(1, D_H),
    ),
    out_shape=(
        jax.ShapeDtypeStruct((N, D_H // 2), jnp.float32),
        jax.ShapeDtypeStruct((N, D_H // 2), jnp.float32),
        jax.ShapeDtypeStruct((1, D_H), jnp.float32),
    ),
)


def _score_body(*refs):
    out = refs[-1]
    acc = jnp.zeros((1, D_OUT), jnp.float32)
    for l in range(NUM_LAYERS):
        cs = refs[l][...]
        w = refs[NUM_LAYERS + l][...]
        b = refs[2 * NUM_LAYERS + l][...]
        acc = acc + jnp.dot(cs, w, preferred_element_type=jnp.float32) + b
    out[...] = acc


def _score_call(dims):
    in_specs = (
        [_full_spec(1, d) for d in dims]
        + [_full_spec(d, D_OUT) for d in dims]
        + [_full_spec(1, D_OUT) for _ in dims]
    )
    return pl.pallas_call(
        _score_body,
        grid=(1,),
        in_specs=in_specs,
        out_specs=_full_spec(1, D_OUT),
        out_shape=jax.ShapeDtypeStruct((1, D_OUT), jnp.float32),
    )


@jax.jit
def kernel(x, edge_index, params):
    row = edge_index[0]
    col = edge_index[1]
    pad = EPAD - E
    colp = jnp.concatenate([col, jnp.zeros((pad,), jnp.int32)]).reshape(EPAD // CHUNK, CHUNK)
    rowp = jnp.concatenate([row, jnp.full((pad,), N, jnp.int32)]).reshape(EPAD // CHUNK, CHUNK)

    colsums = []
    hL = hR = None
    for l in range(NUM_LAYERS - 1):
        p = params["layers"][l]
        c = (1.0 + params["eps"][l]).reshape(1, 1).astype(jnp.float32)
        if l == 0:
            seg0, seg1 = _make_segsum(False)(x, x, colp, rowp)
            a, s1, s2, csx = _t1a_call(
                seg0, seg1, x, p["W1"], p["b1"].reshape(1, D_H), c
            )
            colsums.append(csx)
        else:
            segL, segR = _make_segsum(True)(hL, hR, colp, rowp)
            a, s1, s2, _ = _t1_call(
                segL, segR, hL, hR, p["W1"], p["b1"].reshape(1, D_H), c
            )
        h2, u1, u2 = _t2_call(
            a, s1, s2,
            p["g1"].reshape(1, D_H), p["be1"].reshape(1, D_H),
            p["W2"], p["b2"].reshape(1, D_H),
        )
        hL, hR, cs = _t3_call(
            h2, u1, u2,
            p["g_out"].reshape(1, D_H), p["be_out"].reshape(1, D_H),
        )
        colsums.append(cs)

    dims = [D_IN] + [D_H] * (NUM_LAYERS - 1)
    ws = [params["preds"][l]["W"] for l in range(NUM_LAYERS)]
    bs = [params["preds"][l]["b"].reshape(1, D_OUT) for l in range(NUM_LAYERS)]
    return _score_call(dims)(*(colsums + ws + bs))
